# Initial kernel scaffold; baseline (speedup 1.0000x reference)
import jax
import jax.numpy as jnp
from jax.experimental import pallas as pl


def kernel(x, edge_index, W1, b1, W2, b2):
    raise NotImplementedError("write your pallas kernel here")


# speedup vs baseline: 22.5671x; 22.5671x over previous
import functools

import jax
import jax.numpy as jnp
from jax import lax
from jax.experimental import pallas as pl
from jax.experimental.pallas import tpu as pltpu
from jax.experimental.pallas import tpu_sc as plsc

NUM_CORES = 2
NUM_SUBCORES = 16
NW = NUM_CORES * NUM_SUBCORES
BC = 128

_mesh = plsc.VectorSubcoreMesh(core_axis_name="core", subcore_axis_name="subcore")
_sc_params = pltpu.CompilerParams(use_tc_tiling_on_sc=False)


def _deg_kernel(npad, ch, rpt):
  @functools.partial(
      pl.kernel,
      out_type=jax.ShapeDtypeStruct((NUM_CORES, npad, 16), jnp.float32),
      mesh=_mesh,
      compiler_params=_sc_params,
      scratch_types=[
          pltpu.VMEM((ch, BC), jnp.int32),
          pltpu.VMEM((BC, 16), jnp.float32),
          pltpu.VMEM_SHARED((npad, 16), jnp.float32),
      ],
  )
  def k(dstp_hbm, ones_hbm, zeros_hbm, out_hbm, dst_v, ones_v, acc):
    c = lax.axis_index("core")
    s = lax.axis_index("subcore")
    w = c * NUM_SUBCORES + s
    pltpu.sync_copy(dstp_hbm.at[w], dst_v)
    pltpu.sync_copy(ones_hbm, ones_v)
    pltpu.sync_copy(zeros_hbm, acc.at[pl.ds(s * rpt, rpt)])
    plsc.subcore_barrier()

    @pl.loop(0, ch)
    def _(j):
      pltpu.sync_copy(ones_v, acc.at[dst_v.at[j]], add=True)

    plsc.subcore_barrier()
    pltpu.sync_copy(acc.at[pl.ds(s * rpt, rpt)],
                    out_hbm.at[c, pl.ds(s * rpt, rpt)])

  return k


def _agg_kernel(npad, ch, rpt):
  @functools.partial(
      pl.kernel,
      out_type=jax.ShapeDtypeStruct((NUM_CORES, npad, 16), jnp.float32),
      mesh=_mesh,
      compiler_params=_sc_params,
      scratch_types=[
          pltpu.VMEM((ch, BC), jnp.int32),
          pltpu.VMEM((ch, BC), jnp.int32),
          pltpu.VMEM((BC, 16), jnp.float32),
          pltpu.VMEM((BC, 16), jnp.float32),
          pltpu.VMEM_SHARED((npad, 16), jnp.float32),
          pltpu.SemaphoreType.DMA,
          pltpu.SemaphoreType.DMA,
      ],
  )
  def k(t_hbm, srcp_hbm, dstp_hbm, zeros_hbm, out_hbm,
        src_v, dst_v, buf0, buf1, acc, sem0, sem1):
    c = lax.axis_index("core")
    s = lax.axis_index("subcore")
    w = c * NUM_SUBCORES + s
    pltpu.sync_copy(srcp_hbm.at[w], src_v)
    pltpu.sync_copy(dstp_hbm.at[w], dst_v)
    pltpu.sync_copy(zeros_hbm, acc.at[pl.ds(s * rpt, rpt)])
    plsc.subcore_barrier()

    pltpu.async_copy(t_hbm.at[src_v.at[0]], buf0, sem0)
    pltpu.async_copy(t_hbm.at[src_v.at[1]], buf1, sem1)

    @pl.loop(0, ch // 2)
    def _(t):
      j0 = 2 * t
      j1 = j0 + 1
      pltpu.make_async_copy(t_hbm.at[src_v.at[j0]], buf0, sem0).wait()
      pltpu.sync_copy(buf0, acc.at[dst_v.at[j0]], add=True)

      @pl.when(j0 + 2 < ch)
      def _():
        pltpu.async_copy(t_hbm.at[src_v.at[j0 + 2]], buf0, sem0)

      pltpu.make_async_copy(t_hbm.at[src_v.at[j1]], buf1, sem1).wait()
      pltpu.sync_copy(buf1, acc.at[dst_v.at[j1]], add=True)

      @pl.when(j1 + 2 < ch)
      def _():
        pltpu.async_copy(t_hbm.at[src_v.at[j1 + 2]], buf1, sem1)

    plsc.subcore_barrier()
    pltpu.sync_copy(acc.at[pl.ds(s * rpt, rpt)],
                    out_hbm.at[c, pl.ds(s * rpt, rpt)])

  return k


def kernel(x, edge_index, W1, b1, W2, b2):
  n, d = x.shape
  h = W1.shape[1]
  e = edge_index.shape[1]

  ept = -(-e // (NW * BC)) * BC
  ch = ept // BC
  rpt = -(-(n + 1) // (NUM_SUBCORES * 8)) * 8
  npad = rpt * NUM_SUBCORES

  src = edge_index[0]
  dst = edge_index[1]
  pad = NW * ept - e
  srcp = jnp.concatenate([src, jnp.zeros((pad,), jnp.int32)]).reshape(NW, ch, BC)
  dstp = jnp.concatenate([dst, jnp.full((pad,), n, jnp.int32)]).reshape(NW, ch, BC)
  zeros16 = jnp.zeros((rpt, 16), jnp.float32)
  ones16 = jnp.ones((BC, 16), jnp.float32)
  b1r = b1.reshape(1, h)
  b2r = b2.reshape(1, W2.shape[1])

  deg_k = _deg_kernel(npad, ch, rpt)
  agg_k = _agg_kernel(npad, ch, rpt)

  bn = 2000
  grid = (n // bn,)

  def _k_mm(x_ref, w_ref, o_ref):
    o_ref[...] = jnp.dot(x_ref[...], w_ref[...],
                         preferred_element_type=jnp.float32)

  hh = pl.pallas_call(
      _k_mm,
      grid=grid,
      in_specs=[pl.BlockSpec((bn, d), lambda i: (i, 0)),
                pl.BlockSpec((d, h), lambda i: (0, 0))],
      out_specs=pl.BlockSpec((bn, h), lambda i: (i, 0)),
      out_shape=jax.ShapeDtypeStruct((n, h), jnp.float32),
  )(x, W1)

  degp = deg_k(dstp, ones16, zeros16)
  p0 = degp[0, :n, 0:1]
  p1 = degp[1, :n, 0:1]

  def _k2(p0_ref, p1_ref, hh_ref, t_ref, dis_ref):
    dis = lax.rsqrt(1.0 + p0_ref[...] + p1_ref[...])
    dis_ref[...] = dis
    t_ref[...] = hh_ref[...] * dis

  t1, dis = pl.pallas_call(
      _k2,
      grid=grid,
      in_specs=[pl.BlockSpec((bn, 1), lambda i: (i, 0)),
                pl.BlockSpec((bn, 1), lambda i: (i, 0)),
                pl.BlockSpec((bn, h), lambda i: (i, 0))],
      out_specs=[pl.BlockSpec((bn, h), lambda i: (i, 0)),
                 pl.BlockSpec((bn, 1), lambda i: (i, 0))],
      out_shape=[jax.ShapeDtypeStruct((n, h), jnp.float32),
                 jax.ShapeDtypeStruct((n, 1), jnp.float32)],
  )(p0, p1, hh)

  s1p = agg_k(t1, srcp, dstp, zeros16)

  def _k4(s0_ref, s1_ref, t_ref, dis_ref, b_ref, u_ref):
    agg = (s0_ref[...] + s1_ref[...] + t_ref[...]) * dis_ref[...] + b_ref[...]
    u_ref[...] = jnp.maximum(agg, 0.0) * dis_ref[...]

  u = pl.pallas_call(
      _k4,
      grid=grid,
      in_specs=[pl.BlockSpec((bn, h), lambda i: (i, 0)),
                pl.BlockSpec((bn, h), lambda i: (i, 0)),
                pl.BlockSpec((bn, h), lambda i: (i, 0)),
                pl.BlockSpec((bn, 1), lambda i: (i, 0)),
                pl.BlockSpec((1, h), lambda i: (0, 0))],
      out_specs=pl.BlockSpec((bn, h), lambda i: (i, 0)),
      out_shape=jax.ShapeDtypeStruct((n, h), jnp.float32),
  )(s1p[0, :n], s1p[1, :n], t1, dis, b1r)

  s2p = agg_k(u, srcp, dstp, zeros16)

  c = W2.shape[1]

  def _k6(s0_ref, s1_ref, u_ref, dis_ref, w_ref, b_ref, o_ref):
    agg = (s0_ref[...] + s1_ref[...] + u_ref[...]) * dis_ref[...]
    o_ref[...] = jnp.dot(agg, w_ref[...],
                         preferred_element_type=jnp.float32) + b_ref[...]

  out = pl.pallas_call(
      _k6,
      grid=grid,
      in_specs=[pl.BlockSpec((bn, h), lambda i: (i, 0)),
                pl.BlockSpec((bn, h), lambda i: (i, 0)),
                pl.BlockSpec((bn, h), lambda i: (i, 0)),
                pl.BlockSpec((bn, 1), lambda i: (i, 0)),
                pl.BlockSpec((h, c), lambda i: (0, 0)),
                pl.BlockSpec((1, c), lambda i: (0, 0))],
      out_specs=pl.BlockSpec((bn, c), lambda i: (i, 0)),
      out_shape=jax.ShapeDtypeStruct((n, c), jnp.float32),
  )(s2p[0, :n], s2p[1, :n], u, dis, W2, b2r)

  return out
